# baseline (speedup 1.0000x reference)
import functools

import jax
import jax.numpy as jnp
from jax import lax
from jax.experimental import pallas as pl
from jax.experimental.pallas import tpu as pltpu
from jax.experimental.pallas import tpu_sc as plsc

_B = 8
_N = 2048
_EMB = 128
_MAXD = 25

_ROWS = _B * _N
_BLOCK_ROWS = 512
_GRID = _ROWS // _BLOCK_ROWS

_INFO = plsc.get_sparse_core_info()
_NC = _INFO.num_cores
_NS = _INFO.num_subcores
_NW = _NC * _NS
_RPW = _ROWS // _NW
_CHUNK = 128
_NCHUNK = _RPW // _CHUNK


def _deg_kernel(adj_ref, idx_ref):
    deg = jnp.sum(adj_ref[...], axis=1)
    idx = jnp.minimum(jnp.round(deg), float(_MAXD))
    idx = jnp.maximum(idx, 0.0).astype(jnp.int32)
    idx_ref[...] = idx.reshape(1, 1, _BLOCK_ROWS)


_deg_call = pl.pallas_call(
    _deg_kernel,
    grid=(_GRID,),
    in_specs=[pl.BlockSpec((_BLOCK_ROWS, _N), lambda i: (i, 0))],
    out_specs=pl.BlockSpec((1, 1, _BLOCK_ROWS), lambda i: (i, 0, 0)),
    out_shape=jax.ShapeDtypeStruct((_GRID, 1, _BLOCK_ROWS), jnp.int32),
)


@functools.partial(
    pl.kernel,
    out_type=jax.ShapeDtypeStruct((_ROWS, _EMB), jnp.float32),
    mesh=plsc.VectorSubcoreMesh(core_axis_name="c", subcore_axis_name="s"),
    scratch_types=[
        pltpu.VMEM((_NCHUNK, _CHUNK), jnp.int32),
        pltpu.VMEM((_NCHUNK, _CHUNK, _EMB), jnp.float32),
        pltpu.SemaphoreType.DMA,
    ],
)
def _gather_kernel(table_hbm, idx_hbm, out_hbm, idx_v, rows_v, sem):
    wid = lax.axis_index("s") * _NC + lax.axis_index("c")
    pltpu.sync_copy(idx_hbm.at[wid], idx_v)
    copies = [
        pltpu.async_copy(table_hbm.at[idx_v.at[j]], rows_v.at[j], sem)
        for j in range(_NCHUNK)
    ]
    for c in copies:
        c.wait()
    base = wid * _RPW
    for j in range(_NCHUNK):
        pltpu.sync_copy(rows_v.at[j], out_hbm.at[pl.ds(base + j * _CHUNK, _CHUNK)])


def kernel(data, adj, dense, emb_weight):
    adj_flat = adj.reshape(_ROWS, _N)
    idx = _deg_call(adj_flat)
    idx = idx.reshape(_NW, _NCHUNK, _CHUNK)
    out = _gather_kernel(emb_weight, idx)
    return out.reshape(_B, _N, _EMB)


# speedup vs baseline: 6.8612x; 6.8612x over previous
import functools

import jax
import jax.numpy as jnp
from jax import lax
from jax.experimental import pallas as pl
from jax.experimental.pallas import tpu as pltpu
from jax.experimental.pallas import tpu_sc as plsc

_B = 8
_N = 2048
_EMB = 128
_MAXD = 25

_ROWS = _B * _N
_BLOCK_ROWS = 512
_GRID = _ROWS // _BLOCK_ROWS

_INFO = plsc.get_sparse_core_info()
_NC = _INFO.num_cores
_NS = _INFO.num_subcores
_NW = _NC * _NS
_RPW = _ROWS // _NW
_CHUNK = 128
_NCHUNK = _RPW // _CHUNK


def _deg_kernel(adj_ref, idx_ref):
    deg = jnp.sum(adj_ref[...], axis=1)
    idx = jnp.minimum(jnp.round(deg), float(_MAXD))
    idx = jnp.maximum(idx, 0.0).astype(jnp.int32)
    idx_ref[...] = idx.reshape(1, 1, _BLOCK_ROWS)


_deg_call = pl.pallas_call(
    _deg_kernel,
    grid=(_GRID,),
    in_specs=[pl.BlockSpec((_BLOCK_ROWS, _N), lambda i: (i, 0))],
    out_specs=pl.BlockSpec((1, 1, _BLOCK_ROWS), lambda i: (i, 0, 0)),
    out_shape=jax.ShapeDtypeStruct((_GRID, 1, _BLOCK_ROWS), jnp.int32),
)


@functools.partial(
    pl.kernel,
    out_type=jax.ShapeDtypeStruct((_ROWS, _EMB), jnp.float32),
    mesh=plsc.VectorSubcoreMesh(core_axis_name="c", subcore_axis_name="s"),
    scratch_types=[
        pltpu.VMEM((_NCHUNK, _CHUNK), jnp.int32),
        pltpu.VMEM((_NCHUNK, _CHUNK, _EMB), jnp.float32),
        pltpu.SemaphoreType.DMA,
    ],
)
def _gather_kernel(table_hbm, idx_hbm, out_hbm, idx_v, rows_v, sem):
    wid = lax.axis_index("s") * _NC + lax.axis_index("c")
    pltpu.sync_copy(idx_hbm.at[wid], idx_v)
    copies = [
        pltpu.async_copy(table_hbm.at[idx_v.at[j]], rows_v.at[j], sem)
        for j in range(_NCHUNK)
    ]
    for c in copies:
        c.wait()
    base = wid * _RPW
    for j in range(_NCHUNK):
        pltpu.sync_copy(rows_v.at[j], out_hbm.at[pl.ds(base + j * _CHUNK, _CHUNK)])


def kernel(data, adj, dense, emb_weight):
    adj_flat = adj.reshape(_ROWS, _N)
    idx = _deg_call(adj_flat)
    out = emb_weight[idx.reshape(_ROWS)]
    return out.reshape(_B, _N, _EMB)
